# baseline (speedup 1.0000x reference)
import jax
import jax.numpy as jnp
from jax import lax
from jax.experimental import pallas as pl
from jax.experimental.pallas import tpu as pltpu, tpu_sc as plsc

_MAX_LEN = 8192
_DIM = 768
_BATCH = 2
_NW = 32
_ROWS_PER_W = _MAX_LEN // _NW
_CHUNK_ROWS = 16
_N_CHUNKS = _ROWS_PER_W // _CHUNK_ROWS
_LANE_STEPS = _DIM // 16


def _sc_kernel(x_hbm, pos_hbm, out_hbm,
               x_v0, x_v1, x_v2, x_v3, pos_v0, pos_v1,
               in_s0, in_s1, in_s2, in_s3,
               out_s0, out_s1, out_s2, out_s3,
               pos_s0, pos_s1):
    x_vs = [x_v0, x_v1, x_v2, x_v3]
    pos_vs = [pos_v0, pos_v1]
    in_sems = [in_s0, in_s1, in_s2, in_s3]
    out_sems = [out_s0, out_s1, out_s2, out_s3]
    pos_sems = [pos_s0, pos_s1]

    wid = lax.axis_index("s") * 2 + lax.axis_index("c")
    row0 = wid * _ROWS_PER_W

    def rows(c):
        return pl.ds(row0 + c * _CHUNK_ROWS, _CHUNK_ROWS)

    def x_in(c):
        pltpu.async_copy(x_hbm.at[:, rows(c), :], x_vs[c % 4],
                         in_sems[c % 4])

    def pos_in(c):
        pltpu.async_copy(pos_hbm.at[rows(c), :], pos_vs[c % 2],
                         pos_sems[c % 2])

    pos_in(0)
    pos_in(1)
    x_in(0)
    x_in(1)
    x_in(2)

    for c in range(_N_CHUNKS):
        xb, pb = c % 4, c % 2
        pltpu.make_async_copy(x_hbm.at[:, rows(c), :], x_vs[xb],
                              in_sems[xb]).wait()
        pltpu.make_async_copy(pos_hbm.at[rows(c), :], pos_vs[pb],
                              pos_sems[pb]).wait()

        for b in range(_BATCH):
            def body(r, _, xb=xb, pb=pb, b=b):
                for j in range(_LANE_STEPS):
                    o = j * 16
                    x_vs[xb][b, r, pl.ds(o, 16)] = (
                        x_vs[xb][b, r, pl.ds(o, 16)]
                        + pos_vs[pb][r, pl.ds(o, 16)])
                return 0

            lax.fori_loop(0, _CHUNK_ROWS, body, 0)

        pltpu.async_copy(x_vs[xb], out_hbm.at[:, rows(c), :], out_sems[xb])

        if c + 2 < _N_CHUNKS:
            pos_in(c + 2)

        cn = c + 3
        if cn < _N_CHUNKS:
            if cn >= 4:
                pltpu.make_async_copy(x_vs[cn % 4],
                                      out_hbm.at[:, rows(cn - 4), :],
                                      out_sems[cn % 4]).wait()
            x_in(cn)

    for c in range(_N_CHUNKS - 4, _N_CHUNKS):
        pltpu.make_async_copy(x_vs[c % 4], out_hbm.at[:, rows(c), :],
                              out_sems[c % 4]).wait()


def kernel(x, pos_table):
    batch, max_len, dim = x.shape
    mesh = plsc.VectorSubcoreMesh(core_axis_name="c", subcore_axis_name="s")
    return pl.kernel(
        _sc_kernel,
        mesh=mesh,
        out_type=jax.ShapeDtypeStruct((batch, max_len, dim), jnp.float32),
        scratch_types=(
            [pltpu.VMEM((_BATCH, _CHUNK_ROWS, _DIM), jnp.float32)] * 4
            + [pltpu.VMEM((_CHUNK_ROWS, _DIM), jnp.float32)] * 2
            + [pltpu.SemaphoreType.DMA] * 10
        ),
    )(x, pos_table)


# speedup vs baseline: 1.2979x; 1.2979x over previous
import jax
import jax.numpy as jnp
from jax import lax
from jax.experimental import pallas as pl
from jax.experimental.pallas import tpu as pltpu, tpu_sc as plsc

_MAX_LEN = 8192
_DIM = 768
_BATCH = 2
_NW = 32
_ROWS_PER_W = _MAX_LEN // _NW
_CHUNK_ROWS = 32
_N_CHUNKS = _ROWS_PER_W // _CHUNK_ROWS
_N_ITEMS = _N_CHUNKS * _BATCH
_LANE_STEPS = _DIM // 16


def _sc_kernel(x_hbm, pos_hbm, out_hbm,
               x_v0, x_v1, x_v2, pos_v0, pos_v1,
               in_s0, in_s1, in_s2, out_s0, out_s1, out_s2,
               pos_s0, pos_s1):
    x_vs = [x_v0, x_v1, x_v2]
    pos_vs = [pos_v0, pos_v1]
    in_sems = [in_s0, in_s1, in_s2]
    out_sems = [out_s0, out_s1, out_s2]
    pos_sems = [pos_s0, pos_s1]

    wid = lax.axis_index("s") * 2 + lax.axis_index("c")
    row0 = wid * _ROWS_PER_W

    def x_in(k):
        ci, b, xb = k // 2, k % 2, k % 3
        pltpu.async_copy(
            x_hbm.at[b, pl.ds(row0 + ci * _CHUNK_ROWS, _CHUNK_ROWS), :],
            x_vs[xb], in_sems[xb])

    def pos_in(ci):
        pltpu.async_copy(
            pos_hbm.at[pl.ds(row0 + ci * _CHUNK_ROWS, _CHUNK_ROWS), :],
            pos_vs[ci % 2], pos_sems[ci % 2])

    pos_in(0)
    pos_in(1)
    x_in(0)
    x_in(1)

    for k in range(_N_ITEMS):
        ci, b, xb, pb = k // 2, k % 2, k % 3, (k // 2) % 2
        pltpu.make_async_copy(
            x_hbm.at[b, pl.ds(row0 + ci * _CHUNK_ROWS, _CHUNK_ROWS), :],
            x_vs[xb], in_sems[xb]).wait()
        if b == 0:
            pltpu.make_async_copy(
                pos_hbm.at[pl.ds(row0 + ci * _CHUNK_ROWS, _CHUNK_ROWS), :],
                pos_vs[pb], pos_sems[pb]).wait()

        def body(r, _, xb=xb, pb=pb):
            for j in range(_LANE_STEPS):
                o = j * 16
                x_vs[xb][r, pl.ds(o, 16)] = (x_vs[xb][r, pl.ds(o, 16)]
                                             + pos_vs[pb][r, pl.ds(o, 16)])
            return 0

        lax.fori_loop(0, _CHUNK_ROWS, body, 0)

        pltpu.async_copy(
            x_vs[xb],
            out_hbm.at[b, pl.ds(row0 + ci * _CHUNK_ROWS, _CHUNK_ROWS), :],
            out_sems[xb])

        if b == 1 and ci + 2 < _N_CHUNKS:
            pos_in(ci + 2)

        kn = k + 2
        if kn < _N_ITEMS:
            if kn >= 3:
                cp, bp = (kn - 3) // 2, (kn - 3) % 2
                pltpu.make_async_copy(
                    x_vs[kn % 3],
                    out_hbm.at[bp, pl.ds(row0 + cp * _CHUNK_ROWS,
                                         _CHUNK_ROWS), :],
                    out_sems[kn % 3]).wait()
            x_in(kn)

    for k in range(_N_ITEMS - 3, _N_ITEMS):
        ci, b = k // 2, k % 2
        pltpu.make_async_copy(
            x_vs[k % 3],
            out_hbm.at[b, pl.ds(row0 + ci * _CHUNK_ROWS, _CHUNK_ROWS), :],
            out_sems[k % 3]).wait()


def kernel(x, pos_table):
    batch, max_len, dim = x.shape
    mesh = plsc.VectorSubcoreMesh(core_axis_name="c", subcore_axis_name="s")
    return pl.kernel(
        _sc_kernel,
        mesh=mesh,
        out_type=jax.ShapeDtypeStruct((batch, max_len, dim), jnp.float32),
        scratch_types=(
            [pltpu.VMEM((_CHUNK_ROWS, _DIM), jnp.float32)] * 3
            + [pltpu.VMEM((_CHUNK_ROWS, _DIM), jnp.float32)] * 2
            + [pltpu.SemaphoreType.DMA] * 8
        ),
    )(x, pos_table)
